# baseline (speedup 1.0000x reference)
import functools

import jax
import jax.numpy as jnp
from jax.experimental import pallas as pl

_TOP_K = 8


def _gate_body(x_ref, w_ref, sm_ref, idx_ref):
    logits = jax.lax.dot_general(
        w_ref[...], x_ref[...],
        dimension_numbers=(((1,), (1,)), ((), ())),
        preferred_element_type=jnp.float32,
    )
    ne, bt = logits.shape
    row = jax.lax.broadcasted_iota(jnp.int32, (ne, bt), 0)
    neg_inf = jnp.float32(-jnp.inf)
    vals = logits
    scores = []
    indices = []
    for _ in range(_TOP_K):
        m = jnp.max(vals, axis=0, keepdims=True)
        is_max = vals == m
        ind = jnp.min(jnp.where(is_max, row, ne), axis=0, keepdims=True)
        scores.append(m)
        indices.append(ind)
        vals = jnp.where(is_max & (row == ind), neg_inf, vals)
    s = jnp.concatenate(scores, axis=0)
    idx = jnp.concatenate(indices, axis=0)
    e = jnp.exp(s - s[0:1, :])
    sm_ref[...] = e / jnp.sum(e, axis=0, keepdims=True)
    idx_ref[...] = idx


@functools.partial(jax.jit, static_argnames=("bt",))
def _gate(x, w, bt):
    t, d = x.shape
    ne = w.shape[0]
    return pl.pallas_call(
        _gate_body,
        grid=(t // bt,),
        in_specs=[
            pl.BlockSpec((bt, d), lambda i: (i, 0)),
            pl.BlockSpec((ne, d), lambda i: (0, 0)),
        ],
        out_specs=[
            pl.BlockSpec((_TOP_K, bt), lambda i: (0, i)),
            pl.BlockSpec((_TOP_K, bt), lambda i: (0, i)),
        ],
        out_shape=[
            jax.ShapeDtypeStruct((_TOP_K, t), jnp.float32),
            jax.ShapeDtypeStruct((_TOP_K, t), jnp.int32),
        ],
    )(x, w)


def kernel(x, W):
    smt, idxt = _gate(x, W, bt=1024)
    return (smt.T, idxt.T)
